# baseline (speedup 1.0000x reference)
import functools

import jax
import jax.numpy as jnp
from jax import lax
from jax.experimental import pallas as pl
from jax.experimental.pallas import tpu as pltpu
from jax.experimental.pallas import tpu_sc as plsc

_BATCH = 16384
_DIM = 64
_LANES = 16
_NC = 2
_NS = 16
_NW = _NC * _NS
_B_PER_W = _BATCH // _NW
_CHUNK = 128
_N_CHUNKS = _B_PER_W // _CHUNK

_mesh = plsc.VectorSubcoreMesh(core_axis_name="c", subcore_axis_name="s")


@functools.partial(
    pl.kernel,
    mesh=_mesh,
    out_type=jax.ShapeDtypeStruct((_BATCH, _DIM), jnp.float32),
    scratch_types=[
        pltpu.VMEM((_N_CHUNKS, _CHUNK), jnp.int32),
        pltpu.VMEM((_N_CHUNKS, _CHUNK, _DIM), jnp.float32),
        pltpu.SemaphoreType.DMA,
    ],
    compiler_params=pltpu.CompilerParams(use_tc_tiling_on_sc=False),
)
def _lookup_softsign(ids_hbm, table_hbm, out_hbm, idx_v, rows_v, sem):
    wid = lax.axis_index("s") * _NC + lax.axis_index("c")
    base = wid * _B_PER_W

    copies = []
    for j in range(_N_CHUNKS):
        pltpu.sync_copy(ids_hbm.at[pl.ds(base + j * _CHUNK, _CHUNK)], idx_v.at[j])
        copies.append(pltpu.async_copy(table_hbm.at[idx_v.at[j]], rows_v.at[j], sem))
    for c in copies:
        c.wait()

    def body(r, carry):
        for j in range(_N_CHUNKS):
            for k in range(_DIM // _LANES):
                x = rows_v[j, r, pl.ds(k * _LANES, _LANES)]
                rows_v[j, r, pl.ds(k * _LANES, _LANES)] = x / (1.0 + jnp.abs(x))
        return carry

    lax.fori_loop(0, _CHUNK, body, 0)

    for j in range(_N_CHUNKS):
        pltpu.sync_copy(rows_v.at[j], out_hbm.at[pl.ds(base + j * _CHUNK, _CHUNK)])


def kernel(spk_ids, spk_embed_table):
    return _lookup_softsign(spk_ids, spk_embed_table)


# speedup vs baseline: 1.0332x; 1.0332x over previous
import functools

import jax
import jax.numpy as jnp
from jax import lax
from jax.experimental import pallas as pl
from jax.experimental.pallas import tpu as pltpu
from jax.experimental.pallas import tpu_sc as plsc

_BATCH = 16384
_DIM = 64
_PAD_DIM = 128
_LANES = 16
_NC = 2
_NS = 16
_NW = _NC * _NS
_B_PER_W = _BATCH // _NW
_CHUNK = 128
_N_CHUNKS = _B_PER_W // _CHUNK

_mesh = plsc.VectorSubcoreMesh(core_axis_name="c", subcore_axis_name="s")


@functools.partial(
    pl.kernel,
    mesh=_mesh,
    out_type=jax.ShapeDtypeStruct((_BATCH, _DIM), jnp.float32),
    scratch_types=[
        pltpu.VMEM((_N_CHUNKS, _CHUNK), jnp.int32),
        pltpu.VMEM((_N_CHUNKS, _CHUNK, _PAD_DIM), jnp.float32),
        pltpu.SemaphoreType.DMA,
    ],
    compiler_params=pltpu.CompilerParams(use_tc_tiling_on_sc=False),
)
def _lookup_softsign(ids_hbm, table_hbm, out_hbm, idx_v, rows_v, sem):
    wid = lax.axis_index("s") * _NC + lax.axis_index("c")
    base = wid * _B_PER_W

    copies = []
    for j in range(_N_CHUNKS):
        pltpu.sync_copy(ids_hbm.at[pl.ds(base + j * _CHUNK, _CHUNK)], idx_v.at[j])
        copies.append(pltpu.async_copy(table_hbm.at[idx_v.at[j]], rows_v.at[j], sem))

    for j in range(_N_CHUNKS):
        copies[j].wait()

        def body(r, carry, j=j):
            for k in range(_DIM // _LANES):
                x = rows_v[j, r, pl.ds(k * _LANES, _LANES)]
                rows_v[j, r, pl.ds(k * _LANES, _LANES)] = x / (1.0 + jnp.abs(x))
            return carry

        lax.fori_loop(0, _CHUNK, body, 0)
        pltpu.sync_copy(
            rows_v.at[j, :, pl.ds(0, _DIM)],
            out_hbm.at[pl.ds(base + j * _CHUNK, _CHUNK)],
        )


def kernel(spk_ids, spk_embed_table):
    table_padded = jnp.pad(spk_embed_table, ((0, 0), (0, _PAD_DIM - _DIM)))
    return _lookup_softsign(spk_ids, table_padded)
